# baseline (speedup 1.0000x reference)
import functools

import jax
import jax.numpy as jnp
from jax import lax
from jax.experimental import pallas as pl
from jax.experimental.pallas import tpu as pltpu
from jax.experimental.pallas import tpu_sc as plsc


_CHUNK_ROWS = 32


def _sc_body(pe_hbm, out_hbm, buf, in_sems, out_sems):
    nw = 32
    rows_per_w = pe_hbm.shape[0] // nw
    n = rows_per_w // _CHUNK_ROWS
    wid = lax.axis_index("s") * 2 + lax.axis_index("c")
    base = wid * rows_per_w

    def in_copy(i, slot):
        return pltpu.make_async_copy(
            pe_hbm.at[pl.ds(base + i * _CHUNK_ROWS, _CHUNK_ROWS)],
            buf.at[slot], in_sems.at[slot])

    def out_copy(i, slot):
        return pltpu.make_async_copy(
            buf.at[slot],
            out_hbm.at[pl.ds(base + i * _CHUNK_ROWS, _CHUNK_ROWS)],
            out_sems.at[slot])

    in_copy(0, 0).start()
    for i in range(n):
        slot = i % 2
        in_copy(i, slot).wait()
        out_copy(i, slot).start()
        if i + 1 < n:
            nslot = (i + 1) % 2
            if i - 1 >= 0:
                out_copy(i - 1, nslot).wait()
            in_copy(i + 1, nslot).start()
    out_copy(n - 2, (n - 2) % 2).wait()
    out_copy(n - 1, (n - 1) % 2).wait()


def kernel(x, pe_weight):
    seq_len = x.shape[1]
    hidden = pe_weight.shape[1]
    k = functools.partial(
        pl.kernel,
        mesh=plsc.VectorSubcoreMesh(core_axis_name="c", subcore_axis_name="s"),
        out_type=jax.ShapeDtypeStruct((seq_len, hidden), pe_weight.dtype),
        scratch_types=[
            pltpu.VMEM((2, _CHUNK_ROWS, hidden), pe_weight.dtype),
            pltpu.SemaphoreType.DMA((2,)),
            pltpu.SemaphoreType.DMA((2,)),
        ],
    )(_sc_body)
    out = k(pe_weight)
    return out[None]
